# baseline (speedup 1.0000x reference)
import functools
import jax
import jax.numpy as jnp
from jax import lax
from jax.experimental import pallas as pl
from jax.experimental.pallas import tpu as pltpu
from jax.experimental.pallas import tpu_sc as plsc

N = 50000
NP = 50048
E0 = 800000
C = 128
NTILES = 16
NCH = 392
EP = NTILES * C * NCH
EPX = EP + NTILES * C
HEADS = 4
DH = 16
H = 64
B0 = 4
RPT = NP // NTILES
ZR = RPT // 8


HCH = NCH // 2


def _sc_body(qt, kt, vt, src_h, dst_h, num_h, den_h,
             sbig, dbig, gqA, gkA, gqB, gkB,
             qbA, kbA, vbA, qbB, kbB, vbB, mbA, exbA, mbB, exbB, zb, zd,
             num_sp, den_sp, semA0, semA1, semA2, semB0, semB1, semB2,
             semA3, semB3):
    c = lax.axis_index("c")
    s = lax.axis_index("s")
    zero16 = jnp.zeros((16,), jnp.float32)
    iota16 = lax.iota(jnp.int32, 16)
    row0 = s * RPT

    def zrow(i, _):
        zb[i] = zero16
        return 0
    lax.fori_loop(0, ZR, zrow, 0)

    def zrowd(i, _):
        zd[pl.ds(i * 16, 16)] = zero16
        return 0
    lax.fori_loop(0, RPT // 16, zrowd, 0)

    def pass_body(p, _):
        hoff = ((p // 2) * 4 + 2 * c + (p % 2)) * NP
        for j in range(8):
            pltpu.sync_copy(zb, num_sp.at[pl.ds(row0 + j * ZR, ZR)])
        pltpu.sync_copy(zd, den_sp.at[pl.ds(row0, RPT)])
        plsc.subcore_barrier()

        def issue(j, gq, gk, qb, kb, vb, s0, s1, s2):
            def gidx(g, _):
                sl = pl.ds(g * 16, 16)
                gq[sl] = dbig[j, sl] + hoff
                gk[sl] = sbig[j, sl] + hoff
                return 0
            lax.fori_loop(0, C // 16, gidx, 0)
            return (pltpu.async_copy(qt.at[gq], qb, s0),
                    pltpu.async_copy(kt.at[gk], kb, s1),
                    pltpu.async_copy(vt.at[gk], vb, s2))

        def compute(j, i, qb, kb, vb, mb, exb, sems, cps):
            for cp in cps:
                cp.wait()

            @pl.when(i != 0)
            def _():
                pltpu.make_async_copy(mb, num_sp.at[dbig.at[j]], sems).wait()
                pltpu.make_async_copy(exb, den_sp.at[dbig.at[j]], sems).wait()

            def grp(g, _):
                av = jnp.zeros((16,), jnp.float32)
                for e in range(16):
                    al = jnp.sum(qb[g * 16 + e] * kb[g * 16 + e])
                    av = jnp.where(iota16 == e, al, av)
                ex = jnp.exp(av * 0.25)
                exb[pl.ds(g * 16, 16)] = ex
                for e in range(16):
                    r = g * 16 + e
                    mb[r] = jnp.full((16,), ex[e], jnp.float32) * vb[r]
                return 0
            lax.fori_loop(0, C // 16, grp, 0)
            pltpu.async_copy(mb, num_sp.at[dbig.at[j]], sems, add=True)
            pltpu.async_copy(exb, den_sp.at[dbig.at[j]], sems, add=True)

        for half in range(2):
            hrow = s * NCH + half * HCH
            pltpu.sync_copy(src_h.at[pl.ds(hrow, HCH + 1)], sbig)
            pltpu.sync_copy(dst_h.at[pl.ds(hrow, HCH + 1)], dbig)

            issue(0, gqA, gkA, qbA, kbA, vbA, semA0, semA1, semA2)

            def pair(i, _):
                cpsB = issue(2 * i + 1, gqB, gkB, qbB, kbB, vbB,
                             semB0, semB1, semB2)
                compute(2 * i, i, qbA, kbA, vbA, mbA, exbA, semA3,
                        (pltpu.make_async_copy(qt.at[gqA], qbA, semA0),
                         pltpu.make_async_copy(kt.at[gkA], kbA, semA1),
                         pltpu.make_async_copy(vt.at[gkA], vbA, semA2)))
                issue(2 * i + 2, gqA, gkA, qbA, kbA, vbA, semA0, semA1, semA2)
                compute(2 * i + 1, i, qbB, kbB, vbB, mbB, exbB, semB3, cpsB)
                return 0
            lax.fori_loop(0, HCH // 2, pair, 0)

            pltpu.make_async_copy(qt.at[gqA], qbA, semA0).wait()
            pltpu.make_async_copy(kt.at[gkA], kbA, semA1).wait()
            pltpu.make_async_copy(vt.at[gkA], vbA, semA2).wait()
            pltpu.make_async_copy(mbA, num_sp.at[dbig.at[0]], semA3).wait()
            pltpu.make_async_copy(exbA, den_sp.at[dbig.at[0]], semA3).wait()
            pltpu.make_async_copy(mbB, num_sp.at[dbig.at[0]], semB3).wait()
            pltpu.make_async_copy(exbB, den_sp.at[dbig.at[0]], semB3).wait()
        plsc.subcore_barrier()

        out0 = hoff + row0
        pltpu.sync_copy(num_sp.at[pl.ds(row0, RPT)], num_h.at[pl.ds(out0, RPT)])
        pltpu.sync_copy(den_sp.at[pl.ds(row0, RPT)], den_h.at[pl.ds(out0, RPT)])
        return 0
    lax.fori_loop(0, 2 * B0, pass_body, 0)


_sc_attn = functools.partial(
    pl.kernel,
    mesh=plsc.VectorSubcoreMesh(core_axis_name="c", subcore_axis_name="s"),
    compiler_params=pltpu.CompilerParams(
        needs_layout_passes=False, use_tc_tiling_on_sc=False),
    out_type=(
        jax.ShapeDtypeStruct((B0 * HEADS * NP, DH), jnp.float32),
        jax.ShapeDtypeStruct((B0 * HEADS * NP,), jnp.float32),
    ),
    scratch_types=[
        pltpu.VMEM((HCH + 1, C), jnp.int32),
        pltpu.VMEM((HCH + 1, C), jnp.int32),
        pltpu.VMEM((C,), jnp.int32),
        pltpu.VMEM((C,), jnp.int32),
        pltpu.VMEM((C,), jnp.int32),
        pltpu.VMEM((C,), jnp.int32),
        pltpu.VMEM((C, DH), jnp.float32),
        pltpu.VMEM((C, DH), jnp.float32),
        pltpu.VMEM((C, DH), jnp.float32),
        pltpu.VMEM((C, DH), jnp.float32),
        pltpu.VMEM((C, DH), jnp.float32),
        pltpu.VMEM((C, DH), jnp.float32),
        pltpu.VMEM((C, DH), jnp.float32),
        pltpu.VMEM((C,), jnp.float32),
        pltpu.VMEM((C, DH), jnp.float32),
        pltpu.VMEM((C,), jnp.float32),
        pltpu.VMEM((ZR, DH), jnp.float32),
        pltpu.VMEM((RPT,), jnp.float32),
        pltpu.VMEM_SHARED((NP, DH), jnp.float32),
        pltpu.VMEM_SHARED((NP,), jnp.float32),
        pltpu.SemaphoreType.DMA,
        pltpu.SemaphoreType.DMA,
        pltpu.SemaphoreType.DMA,
        pltpu.SemaphoreType.DMA,
        pltpu.SemaphoreType.DMA,
        pltpu.SemaphoreType.DMA,
        pltpu.SemaphoreType.DMA,
        pltpu.SemaphoreType.DMA,
    ],
)(_sc_body)


BLK = 1000


def _proj_body(x_ref, w_ref, b_ref, o_ref):
    o_ref[0] = jnp.dot(x_ref[0], w_ref[...], preferred_element_type=jnp.float32) + b_ref[...]


def _tc_proj(xp, w, b):
    B, n, kdim = xp.shape
    hdim = w.shape[1]
    return pl.pallas_call(
        _proj_body,
        grid=(B, n // BLK),
        in_specs=[
            pl.BlockSpec((1, BLK, kdim), lambda b_, i: (b_, i, 0)),
            pl.BlockSpec((kdim, hdim), lambda b_, i: (0, 0)),
            pl.BlockSpec((hdim,), lambda b_, i: (0,)),
        ],
        out_specs=pl.BlockSpec((1, BLK, hdim), lambda b_, i: (b_, i, 0)),
        out_shape=jax.ShapeDtypeStruct((B, n, hdim), jnp.float32),
    )(xp, w, b)


def _qkvs_body(h_ref, wq_ref, bq_ref, wk_ref, bk_ref, wv_ref, bv_ref, ws_ref, bs_ref,
               q_ref, k_ref, v_ref, s_ref):
    hb = h_ref[0]
    q_ref[0] = jnp.dot(hb, wq_ref[...], preferred_element_type=jnp.float32) + bq_ref[...]
    k_ref[0] = jnp.dot(hb, wk_ref[...], preferred_element_type=jnp.float32) + bk_ref[...]
    v_ref[0] = jnp.dot(hb, wv_ref[...], preferred_element_type=jnp.float32) + bv_ref[...]
    s_ref[0] = jnp.dot(hb, ws_ref[...], preferred_element_type=jnp.float32) + bs_ref[...]


def _tc_qkvs(h, wq, bq, wk, bk, wv, bv, ws, bs):
    B, n, _ = h.shape
    wspec = pl.BlockSpec((H, H), lambda b_, i: (0, 0))
    bspec = pl.BlockSpec((H,), lambda b_, i: (0,))
    hspec = pl.BlockSpec((1, BLK, H), lambda b_, i: (b_, i, 0))
    out_sds = jax.ShapeDtypeStruct((B, n, H), jnp.float32)
    return pl.pallas_call(
        _qkvs_body,
        grid=(B, n // BLK),
        in_specs=[hspec, wspec, bspec, wspec, bspec, wspec, bspec, wspec, bspec],
        out_specs=(hspec, hspec, hspec, hspec),
        out_shape=(out_sds, out_sds, out_sds, out_sds),
    )(h, wq, bq, wk, bk, wv, bv, ws, bs)


def _combine_body(num_ref, den_ref, s_ref, h_ref, g_ref, b_ref, o_ref):
    a = num_ref[0] / jnp.maximum(den_ref[0], 1e-16) + s_ref[0] + h_ref[0]
    mu = jnp.mean(a, axis=-1, keepdims=True)
    var = jnp.mean((a - mu) * (a - mu), axis=-1, keepdims=True)
    o_ref[0] = (a - mu) / jnp.sqrt(var + 1e-5) * g_ref[...] + b_ref[...]


def _tc_combine(num_t, den_r, s_, h, gamma, beta):
    B, n, _ = h.shape
    hspec = pl.BlockSpec((1, BLK, H), lambda b_, i: (b_, i, 0))
    vspec = pl.BlockSpec((H,), lambda b_, i: (0,))
    return pl.pallas_call(
        _combine_body,
        grid=(B, n // BLK),
        in_specs=[hspec, hspec, hspec, hspec, vspec, vspec],
        out_specs=hspec,
        out_shape=jax.ShapeDtypeStruct((B, n, H), jnp.float32),
    )(num_t, den_r, s_, h, gamma, beta)


def _headmajor(a):
    t = a.reshape(B0, N, HEADS, DH).transpose(0, 2, 1, 3)
    t = jnp.pad(t, ((0, 0), (0, 0), (0, NP - N), (0, 0)))
    return t.reshape(B0 * HEADS * NP, DH)


def kernel(x, edge_index, W_in, b_in, Wq, bq, Wk, bk, Wv, bv, Ws, bs, gamma, beta, W_out, b_out):
    B = x.shape[0]
    S = x.shape[1]
    L = Wq.shape[0]
    src = edge_index[0]
    dst = edge_index[1]
    srcp = jnp.full((EPX,), N, jnp.int32).at[:E0].set(src).reshape(EPX // C, C)
    dstp = jnp.full((EPX,), N, jnp.int32).at[:E0].set(dst).reshape(EPX // C, C)

    x_t = jnp.transpose(x, (0, 2, 1))
    x_p = jnp.pad(x_t, ((0, 0), (0, 0), (0, 16 - S)))
    W_in_p = jnp.pad(W_in, ((0, 16 - S), (0, 0)))
    h = _tc_proj(x_p, W_in_p, b_in)

    for l in range(L):
        q, k, v, s_ = _tc_qkvs(h, Wq[l], bq[l], Wk[l], bk[l], Wv[l], bv[l], Ws[l], bs[l])
        numb, denb = _sc_attn(_headmajor(q), _headmajor(k), _headmajor(v), srcp, dstp)
        num_t = (numb.reshape(B0, HEADS, NP, DH)[:, :, :N]
                 .transpose(0, 2, 1, 3).reshape(B0, N, H))
        den_r = jnp.repeat(
            denb.reshape(B0, HEADS, NP)[:, :, :N].transpose(0, 2, 1), DH, axis=2)
        h = _tc_combine(num_t, den_r, s_, h, gamma, beta)

    P = W_out.shape[1]
    Wp = jnp.zeros((H, 128), jnp.float32).at[:, :P].set(W_out)
    bp = jnp.zeros((128,), jnp.float32).at[:P].set(b_out)
    out = _tc_proj(h, Wp, bp)
    return jnp.transpose(out[:, :, :P], (0, 2, 1))


# speedup vs baseline: 1.0802x; 1.0802x over previous
import functools
import jax
import jax.numpy as jnp
from jax import lax
from jax.experimental import pallas as pl
from jax.experimental.pallas import tpu as pltpu
from jax.experimental.pallas import tpu_sc as plsc

N = 50000
NP = 50048
E0 = 800000
C = 128
NTILES = 16
NCH = 392
EP = NTILES * C * NCH
EPX = EP + NTILES * C
HEADS = 4
DH = 16
H = 64
B0 = 4
RPT = NP // NTILES
ZR = RPT // 8


HCH = NCH // 2


def _sc_body(qt, kt, vt, src_h, dst_h, num_h, den_h,
             sbig, dbig, gqA, gkA, gqB, gkB,
             qbA, kbA, vbA, qbB, kbB, vbB, mbA, exbA, mbB, exbB, zb, zd,
             num_sp, den_sp, semA0, semA1, semA2, semB0, semB1, semB2,
             semA3, semB3):
    c = lax.axis_index("c")
    s = lax.axis_index("s")
    zero16 = jnp.zeros((16,), jnp.float32)
    iota16 = lax.iota(jnp.int32, 16)
    row0 = s * RPT

    def zrow(i, _):
        zb[i] = zero16
        return 0
    lax.fori_loop(0, ZR, zrow, 0)

    def zrowd(i, _):
        zd[pl.ds(i * 16, 16)] = zero16
        return 0
    lax.fori_loop(0, RPT // 16, zrowd, 0)

    for hl in range(2):
        hoff = (2 * c + hl) * NP
        for j in range(8):
            pltpu.sync_copy(zb, num_sp.at[pl.ds(row0 + j * ZR, ZR)])
        pltpu.sync_copy(zd, den_sp.at[pl.ds(row0, RPT)])
        plsc.subcore_barrier()

        def issue(j, gq, gk, qb, kb, vb, s0, s1, s2):
            def gidx(g, _):
                sl = pl.ds(g * 16, 16)
                gq[sl] = dbig[j, sl] + hoff
                gk[sl] = sbig[j, sl] + hoff
                return 0
            lax.fori_loop(0, C // 16, gidx, 0)
            return (pltpu.async_copy(qt.at[gq], qb, s0),
                    pltpu.async_copy(kt.at[gk], kb, s1),
                    pltpu.async_copy(vt.at[gk], vb, s2))

        def compute(j, i, qb, kb, vb, mb, exb, sems, cps):
            for cp in cps:
                cp.wait()

            @pl.when(i != 0)
            def _():
                pltpu.make_async_copy(mb, num_sp.at[dbig.at[j]], sems).wait()
                pltpu.make_async_copy(exb, den_sp.at[dbig.at[j]], sems).wait()

            def grp(g, _):
                av = jnp.zeros((16,), jnp.float32)
                for e in range(16):
                    al = jnp.sum(qb[g * 16 + e] * kb[g * 16 + e])
                    av = jnp.where(iota16 == e, al, av)
                ex = jnp.exp(av * 0.25)
                exb[pl.ds(g * 16, 16)] = ex
                for e in range(16):
                    r = g * 16 + e
                    mb[r] = jnp.full((16,), ex[e], jnp.float32) * vb[r]
                return 0
            lax.fori_loop(0, C // 16, grp, 0)
            pltpu.async_copy(mb, num_sp.at[dbig.at[j]], sems, add=True)
            pltpu.async_copy(exb, den_sp.at[dbig.at[j]], sems, add=True)

        for half in range(2):
            hrow = s * NCH + half * HCH
            pltpu.sync_copy(src_h.at[pl.ds(hrow, HCH + 1)], sbig)
            pltpu.sync_copy(dst_h.at[pl.ds(hrow, HCH + 1)], dbig)

            issue(0, gqA, gkA, qbA, kbA, vbA, semA0, semA1, semA2)

            def pair(i, _):
                cpsB = issue(2 * i + 1, gqB, gkB, qbB, kbB, vbB,
                             semB0, semB1, semB2)
                compute(2 * i, i, qbA, kbA, vbA, mbA, exbA, semA3,
                        (pltpu.make_async_copy(qt.at[gqA], qbA, semA0),
                         pltpu.make_async_copy(kt.at[gkA], kbA, semA1),
                         pltpu.make_async_copy(vt.at[gkA], vbA, semA2)))
                issue(2 * i + 2, gqA, gkA, qbA, kbA, vbA, semA0, semA1, semA2)
                compute(2 * i + 1, i, qbB, kbB, vbB, mbB, exbB, semB3, cpsB)
                return 0
            lax.fori_loop(0, HCH // 2, pair, 0)

            pltpu.make_async_copy(qt.at[gqA], qbA, semA0).wait()
            pltpu.make_async_copy(kt.at[gkA], kbA, semA1).wait()
            pltpu.make_async_copy(vt.at[gkA], vbA, semA2).wait()
            pltpu.make_async_copy(mbA, num_sp.at[dbig.at[0]], semA3).wait()
            pltpu.make_async_copy(exbA, den_sp.at[dbig.at[0]], semA3).wait()
            pltpu.make_async_copy(mbB, num_sp.at[dbig.at[0]], semB3).wait()
            pltpu.make_async_copy(exbB, den_sp.at[dbig.at[0]], semB3).wait()
        plsc.subcore_barrier()

        out0 = hoff + row0
        pltpu.sync_copy(num_sp.at[pl.ds(row0, RPT)], num_h.at[pl.ds(out0, RPT)])
        pltpu.sync_copy(den_sp.at[pl.ds(row0, RPT)], den_h.at[pl.ds(out0, RPT)])


_sc_attn = functools.partial(
    pl.kernel,
    mesh=plsc.VectorSubcoreMesh(core_axis_name="c", subcore_axis_name="s"),
    compiler_params=pltpu.CompilerParams(
        needs_layout_passes=False, use_tc_tiling_on_sc=False),
    out_type=(
        jax.ShapeDtypeStruct((HEADS * NP, DH), jnp.float32),
        jax.ShapeDtypeStruct((HEADS * NP,), jnp.float32),
    ),
    scratch_types=[
        pltpu.VMEM((HCH + 1, C), jnp.int32),
        pltpu.VMEM((HCH + 1, C), jnp.int32),
        pltpu.VMEM((C,), jnp.int32),
        pltpu.VMEM((C,), jnp.int32),
        pltpu.VMEM((C,), jnp.int32),
        pltpu.VMEM((C,), jnp.int32),
        pltpu.VMEM((C, DH), jnp.float32),
        pltpu.VMEM((C, DH), jnp.float32),
        pltpu.VMEM((C, DH), jnp.float32),
        pltpu.VMEM((C, DH), jnp.float32),
        pltpu.VMEM((C, DH), jnp.float32),
        pltpu.VMEM((C, DH), jnp.float32),
        pltpu.VMEM((C, DH), jnp.float32),
        pltpu.VMEM((C,), jnp.float32),
        pltpu.VMEM((C, DH), jnp.float32),
        pltpu.VMEM((C,), jnp.float32),
        pltpu.VMEM((ZR, DH), jnp.float32),
        pltpu.VMEM((RPT,), jnp.float32),
        pltpu.VMEM_SHARED((NP, DH), jnp.float32),
        pltpu.VMEM_SHARED((NP,), jnp.float32),
        pltpu.SemaphoreType.DMA,
        pltpu.SemaphoreType.DMA,
        pltpu.SemaphoreType.DMA,
        pltpu.SemaphoreType.DMA,
        pltpu.SemaphoreType.DMA,
        pltpu.SemaphoreType.DMA,
        pltpu.SemaphoreType.DMA,
        pltpu.SemaphoreType.DMA,
    ],
)(_sc_body)


BLK = 1000


def _proj_body(x_ref, w_ref, b_ref, o_ref):
    o_ref[0] = jnp.dot(x_ref[0], w_ref[...], preferred_element_type=jnp.float32) + b_ref[...]


def _tc_proj(xp, w, b):
    B, n, kdim = xp.shape
    hdim = w.shape[1]
    return pl.pallas_call(
        _proj_body,
        grid=(B, n // BLK),
        in_specs=[
            pl.BlockSpec((1, BLK, kdim), lambda b_, i: (b_, i, 0)),
            pl.BlockSpec((kdim, hdim), lambda b_, i: (0, 0)),
            pl.BlockSpec((hdim,), lambda b_, i: (0,)),
        ],
        out_specs=pl.BlockSpec((1, BLK, hdim), lambda b_, i: (b_, i, 0)),
        out_shape=jax.ShapeDtypeStruct((B, n, hdim), jnp.float32),
    )(xp, w, b)


def _qkvs_body(h_ref, wq_ref, bq_ref, wk_ref, bk_ref, wv_ref, bv_ref, ws_ref, bs_ref,
               q_ref, k_ref, v_ref, s_ref):
    hb = h_ref[0]
    q_ref[0] = jnp.dot(hb, wq_ref[...], preferred_element_type=jnp.float32) + bq_ref[...]
    k_ref[0] = jnp.dot(hb, wk_ref[...], preferred_element_type=jnp.float32) + bk_ref[...]
    v_ref[0] = jnp.dot(hb, wv_ref[...], preferred_element_type=jnp.float32) + bv_ref[...]
    s_ref[0] = jnp.dot(hb, ws_ref[...], preferred_element_type=jnp.float32) + bs_ref[...]


def _tc_qkvs(h, wq, bq, wk, bk, wv, bv, ws, bs):
    B, n, _ = h.shape
    wspec = pl.BlockSpec((H, H), lambda b_, i: (0, 0))
    bspec = pl.BlockSpec((H,), lambda b_, i: (0,))
    hspec = pl.BlockSpec((1, BLK, H), lambda b_, i: (b_, i, 0))
    out_sds = jax.ShapeDtypeStruct((B, n, H), jnp.float32)
    return pl.pallas_call(
        _qkvs_body,
        grid=(B, n // BLK),
        in_specs=[hspec, wspec, bspec, wspec, bspec, wspec, bspec, wspec, bspec],
        out_specs=(hspec, hspec, hspec, hspec),
        out_shape=(out_sds, out_sds, out_sds, out_sds),
    )(h, wq, bq, wk, bk, wv, bv, ws, bs)


def _combine_body(num_ref, den_ref, s_ref, h_ref, g_ref, b_ref, o_ref):
    a = num_ref[0] / jnp.maximum(den_ref[0], 1e-16) + s_ref[0] + h_ref[0]
    mu = jnp.mean(a, axis=-1, keepdims=True)
    var = jnp.mean((a - mu) * (a - mu), axis=-1, keepdims=True)
    o_ref[0] = (a - mu) / jnp.sqrt(var + 1e-5) * g_ref[...] + b_ref[...]


def _tc_combine(num_t, den_r, s_, h, gamma, beta):
    B, n, _ = h.shape
    hspec = pl.BlockSpec((1, BLK, H), lambda b_, i: (b_, i, 0))
    vspec = pl.BlockSpec((H,), lambda b_, i: (0,))
    return pl.pallas_call(
        _combine_body,
        grid=(B, n // BLK),
        in_specs=[hspec, hspec, hspec, hspec, vspec, vspec],
        out_specs=hspec,
        out_shape=jax.ShapeDtypeStruct((B, n, H), jnp.float32),
    )(num_t, den_r, s_, h, gamma, beta)


def _headmajor(a):
    t = a.reshape(N, HEADS, DH).transpose(1, 0, 2)
    t = jnp.pad(t, ((0, 0), (0, NP - N), (0, 0)))
    return t.reshape(HEADS * NP, DH)


def kernel(x, edge_index, W_in, b_in, Wq, bq, Wk, bk, Wv, bv, Ws, bs, gamma, beta, W_out, b_out):
    B = x.shape[0]
    S = x.shape[1]
    L = Wq.shape[0]
    src = edge_index[0]
    dst = edge_index[1]
    srcp = jnp.full((EPX,), N, jnp.int32).at[:E0].set(src).reshape(EPX // C, C)
    dstp = jnp.full((EPX,), N, jnp.int32).at[:E0].set(dst).reshape(EPX // C, C)

    x_t = jnp.transpose(x, (0, 2, 1))
    x_p = jnp.pad(x_t, ((0, 0), (0, 0), (0, 16 - S)))
    W_in_p = jnp.pad(W_in, ((0, 16 - S), (0, 0)))
    h = _tc_proj(x_p, W_in_p, b_in)

    for l in range(L):
        q, k, v, s_ = _tc_qkvs(h, Wq[l], bq[l], Wk[l], bk[l], Wv[l], bv[l], Ws[l], bs[l])
        nums = []
        dens = []
        for b in range(B):
            qt = _headmajor(q[b])
            kt = _headmajor(k[b])
            vt = _headmajor(v[b])
            numb, denb = _sc_attn(qt, kt, vt, srcp, dstp)
            num_t = numb.reshape(HEADS, NP, DH)[:, :N].transpose(1, 0, 2).reshape(N, H)
            den_bn = denb.reshape(HEADS, NP)[:, :N]
            den_r = jnp.repeat(den_bn.T, DH, axis=1)
            nums.append(num_t)
            dens.append(den_r)
        num_t = jnp.stack(nums)
        den_r = jnp.stack(dens)
        h = _tc_combine(num_t, den_r, s_, h, gamma, beta)

    P = W_out.shape[1]
    Wp = jnp.zeros((H, 128), jnp.float32).at[:, :P].set(W_out)
    bp = jnp.zeros((128,), jnp.float32).at[:P].set(b_out)
    out = _tc_proj(h, Wp, bp)
    return jnp.transpose(out[:, :, :P], (0, 2, 1))


# speedup vs baseline: 1.1227x; 1.0394x over previous
import functools
import jax
import jax.numpy as jnp
from jax import lax
from jax.experimental import pallas as pl
from jax.experimental.pallas import tpu as pltpu
from jax.experimental.pallas import tpu_sc as plsc

N = 50000
NP = 50048
E0 = 800000
C = 128
NTILES = 16
NCH = 392
EP = NTILES * C * NCH
EPX = EP + NTILES * C
HEADS = 4
DH = 16
H = 64
B0 = 4
RPT = NP // NTILES
ZR = RPT // 8


HCH = NCH // 2


def _sc_body(qt, kt, vt, src_h, dst_h, num_h, den_h,
             sbig, dbig, gqA, gkA, gqB, gkB,
             qbA, kbA, vbA, qbB, kbB, vbB, mbA, exbA, mbB, exbB, zb, zd,
             num_sp, den_sp, semA0, semA1, semA2, semB0, semB1, semB2,
             semA3, semB3):
    c = lax.axis_index("c")
    s = lax.axis_index("s")
    zero16 = jnp.zeros((16,), jnp.float32)
    iota16 = lax.iota(jnp.int32, 16)
    row0 = s * RPT

    def zrow(i, _):
        zb[i] = zero16
        return 0
    lax.fori_loop(0, ZR, zrow, 0)

    def zrowd(i, _):
        zd[pl.ds(i * 16, 16)] = zero16
        return 0
    lax.fori_loop(0, RPT // 16, zrowd, 0)

    for hl in range(2):
        hoff = (2 * c + hl) * NP
        for j in range(8):
            pltpu.sync_copy(zb, num_sp.at[pl.ds(row0 + j * ZR, ZR)])
        pltpu.sync_copy(zd, den_sp.at[pl.ds(row0, RPT)])
        plsc.subcore_barrier()

        def issue(j, gq, gk, qb, kb, vb, s0, s1, s2):
            def gidx(g, _):
                sl = pl.ds(g * 16, 16)
                gq[sl] = dbig[j, sl] + hoff
                gk[sl] = sbig[j, sl] + hoff
                return 0
            lax.fori_loop(0, C // 16, gidx, 0)
            return (pltpu.async_copy(qt.at[gq], qb, s0),
                    pltpu.async_copy(kt.at[gk], kb, s1),
                    pltpu.async_copy(vt.at[gk], vb, s2))

        def compute(j, i, qb, kb, vb, mb, exb, sems, cps):
            for cp in cps:
                cp.wait()

            @pl.when(i != 0)
            def _():
                pltpu.make_async_copy(mb, num_sp.at[dbig.at[j]], sems).wait()
                pltpu.make_async_copy(exb, den_sp.at[dbig.at[j]], sems).wait()

            def grp(g, _):
                av = jnp.zeros((16,), jnp.float32)
                for e in range(16):
                    al = jnp.sum(qb[g * 16 + e] * kb[g * 16 + e])
                    av = jnp.where(iota16 == e, al, av)
                ex = jnp.exp(av * 0.25)
                exb[pl.ds(g * 16, 16)] = ex
                for e in range(16):
                    r = g * 16 + e
                    mb[r] = jnp.full((16,), ex[e], jnp.float32) * vb[r]
                return 0
            lax.fori_loop(0, C // 16, grp, 0)
            pltpu.async_copy(mb, num_sp.at[dbig.at[j]], sems, add=True)
            pltpu.async_copy(exb, den_sp.at[dbig.at[j]], sems, add=True)

        for half in range(2):
            hrow = s * NCH + half * HCH
            pltpu.sync_copy(src_h.at[pl.ds(hrow, HCH + 1)], sbig)
            pltpu.sync_copy(dst_h.at[pl.ds(hrow, HCH + 1)], dbig)

            issue(0, gqA, gkA, qbA, kbA, vbA, semA0, semA1, semA2)

            def pair(i, _):
                cpsB = issue(2 * i + 1, gqB, gkB, qbB, kbB, vbB,
                             semB0, semB1, semB2)
                compute(2 * i, i, qbA, kbA, vbA, mbA, exbA, semA3,
                        (pltpu.make_async_copy(qt.at[gqA], qbA, semA0),
                         pltpu.make_async_copy(kt.at[gkA], kbA, semA1),
                         pltpu.make_async_copy(vt.at[gkA], vbA, semA2)))
                issue(2 * i + 2, gqA, gkA, qbA, kbA, vbA, semA0, semA1, semA2)
                compute(2 * i + 1, i, qbB, kbB, vbB, mbB, exbB, semB3, cpsB)
                return 0
            lax.fori_loop(0, HCH // 2, pair, 0)

            pltpu.make_async_copy(qt.at[gqA], qbA, semA0).wait()
            pltpu.make_async_copy(kt.at[gkA], kbA, semA1).wait()
            pltpu.make_async_copy(vt.at[gkA], vbA, semA2).wait()
            pltpu.make_async_copy(mbA, num_sp.at[dbig.at[0]], semA3).wait()
            pltpu.make_async_copy(exbA, den_sp.at[dbig.at[0]], semA3).wait()
            pltpu.make_async_copy(mbB, num_sp.at[dbig.at[0]], semB3).wait()
            pltpu.make_async_copy(exbB, den_sp.at[dbig.at[0]], semB3).wait()
        plsc.subcore_barrier()

        out0 = hoff + row0
        pltpu.sync_copy(num_sp.at[pl.ds(row0, RPT)], num_h.at[pl.ds(out0, RPT)])
        pltpu.sync_copy(den_sp.at[pl.ds(row0, RPT)], den_h.at[pl.ds(out0, RPT)])


_sc_attn = functools.partial(
    pl.kernel,
    mesh=plsc.VectorSubcoreMesh(core_axis_name="c", subcore_axis_name="s"),
    compiler_params=pltpu.CompilerParams(
        needs_layout_passes=False, use_tc_tiling_on_sc=False),
    out_type=(
        jax.ShapeDtypeStruct((HEADS * NP, DH), jnp.float32),
        jax.ShapeDtypeStruct((HEADS * NP,), jnp.float32),
    ),
    scratch_types=[
        pltpu.VMEM((HCH + 1, C), jnp.int32),
        pltpu.VMEM((HCH + 1, C), jnp.int32),
        pltpu.VMEM((C,), jnp.int32),
        pltpu.VMEM((C,), jnp.int32),
        pltpu.VMEM((C,), jnp.int32),
        pltpu.VMEM((C,), jnp.int32),
        pltpu.VMEM((C, DH), jnp.float32),
        pltpu.VMEM((C, DH), jnp.float32),
        pltpu.VMEM((C, DH), jnp.float32),
        pltpu.VMEM((C, DH), jnp.float32),
        pltpu.VMEM((C, DH), jnp.float32),
        pltpu.VMEM((C, DH), jnp.float32),
        pltpu.VMEM((C, DH), jnp.float32),
        pltpu.VMEM((C,), jnp.float32),
        pltpu.VMEM((C, DH), jnp.float32),
        pltpu.VMEM((C,), jnp.float32),
        pltpu.VMEM((ZR, DH), jnp.float32),
        pltpu.VMEM((RPT,), jnp.float32),
        pltpu.VMEM_SHARED((NP, DH), jnp.float32),
        pltpu.VMEM_SHARED((NP,), jnp.float32),
        pltpu.SemaphoreType.DMA,
        pltpu.SemaphoreType.DMA,
        pltpu.SemaphoreType.DMA,
        pltpu.SemaphoreType.DMA,
        pltpu.SemaphoreType.DMA,
        pltpu.SemaphoreType.DMA,
        pltpu.SemaphoreType.DMA,
        pltpu.SemaphoreType.DMA,
    ],
)(_sc_body)


BLK = 1000


def _proj_body(x_ref, w_ref, b_ref, o_ref):
    o_ref[0] = jnp.dot(x_ref[0], w_ref[...], preferred_element_type=jnp.float32) + b_ref[...]


def _tc_proj(xp, w, b):
    B, n, kdim = xp.shape
    hdim = w.shape[1]
    return pl.pallas_call(
        _proj_body,
        grid=(B, n // BLK),
        in_specs=[
            pl.BlockSpec((1, BLK, kdim), lambda b_, i: (b_, i, 0)),
            pl.BlockSpec((kdim, hdim), lambda b_, i: (0, 0)),
            pl.BlockSpec((hdim,), lambda b_, i: (0,)),
        ],
        out_specs=pl.BlockSpec((1, BLK, hdim), lambda b_, i: (b_, i, 0)),
        out_shape=jax.ShapeDtypeStruct((B, n, hdim), jnp.float32),
    )(xp, w, b)


def _qkvs_body(h_ref, wq_ref, bq_ref, wk_ref, bk_ref, wv_ref, bv_ref, ws_ref, bs_ref,
               q_ref, k_ref, v_ref, s_ref):
    hb = h_ref[0]
    q = jnp.dot(hb, wq_ref[...], preferred_element_type=jnp.float32) + bq_ref[...]
    k = jnp.dot(hb, wk_ref[...], preferred_element_type=jnp.float32) + bk_ref[...]
    v = jnp.dot(hb, wv_ref[...], preferred_element_type=jnp.float32) + bv_ref[...]
    for hh in range(HEADS):
        q_ref[0, hh] = q[:, hh * DH:(hh + 1) * DH]
        k_ref[0, hh] = k[:, hh * DH:(hh + 1) * DH]
        v_ref[0, hh] = v[:, hh * DH:(hh + 1) * DH]
    s_ref[0] = jnp.dot(hb, ws_ref[...], preferred_element_type=jnp.float32) + bs_ref[...]


def _tc_qkvs(h, wq, bq, wk, bk, wv, bv, ws, bs):
    B, n, _ = h.shape
    wspec = pl.BlockSpec((H, H), lambda b_, i: (0, 0))
    bspec = pl.BlockSpec((H,), lambda b_, i: (0,))
    hspec = pl.BlockSpec((1, BLK, H), lambda b_, i: (b_, i, 0))
    tspec = pl.BlockSpec((1, HEADS, BLK, DH), lambda b_, i: (b_, 0, i, 0))
    t_sds = jax.ShapeDtypeStruct((B, HEADS, NP, DH), jnp.float32)
    out_sds = jax.ShapeDtypeStruct((B, n, H), jnp.float32)
    return pl.pallas_call(
        _qkvs_body,
        grid=(B, n // BLK),
        in_specs=[hspec, wspec, bspec, wspec, bspec, wspec, bspec, wspec, bspec],
        out_specs=(tspec, tspec, tspec, hspec),
        out_shape=(t_sds, t_sds, t_sds, out_sds),
    )(h, wq, bq, wk, bk, wv, bv, ws, bs)


def _combine_body(num_ref, den_ref, s_ref, h_ref, g_ref, b_ref, o_ref):
    a = num_ref[0] / jnp.maximum(den_ref[0], 1e-16) + s_ref[0] + h_ref[0]
    mu = jnp.mean(a, axis=-1, keepdims=True)
    var = jnp.mean((a - mu) * (a - mu), axis=-1, keepdims=True)
    o_ref[0] = (a - mu) / jnp.sqrt(var + 1e-5) * g_ref[...] + b_ref[...]


def _tc_combine(num_t, den_r, s_, h, gamma, beta):
    B, n, _ = h.shape
    hspec = pl.BlockSpec((1, BLK, H), lambda b_, i: (b_, i, 0))
    vspec = pl.BlockSpec((H,), lambda b_, i: (0,))
    return pl.pallas_call(
        _combine_body,
        grid=(B, n // BLK),
        in_specs=[hspec, hspec, hspec, hspec, vspec, vspec],
        out_specs=hspec,
        out_shape=jax.ShapeDtypeStruct((B, n, H), jnp.float32),
    )(num_t, den_r, s_, h, gamma, beta)


def _headmajor(a):
    t = a.reshape(N, HEADS, DH).transpose(1, 0, 2)
    t = jnp.pad(t, ((0, 0), (0, NP - N), (0, 0)))
    return t.reshape(HEADS * NP, DH)


def kernel(x, edge_index, W_in, b_in, Wq, bq, Wk, bk, Wv, bv, Ws, bs, gamma, beta, W_out, b_out):
    B = x.shape[0]
    S = x.shape[1]
    L = Wq.shape[0]
    src = edge_index[0]
    dst = edge_index[1]
    srcp = jnp.full((EPX,), N, jnp.int32).at[:E0].set(src).reshape(EPX // C, C)
    dstp = jnp.full((EPX,), N, jnp.int32).at[:E0].set(dst).reshape(EPX // C, C)

    x_t = jnp.transpose(x, (0, 2, 1))
    x_p = jnp.pad(x_t, ((0, 0), (0, 0), (0, 16 - S)))
    W_in_p = jnp.pad(W_in, ((0, 16 - S), (0, 0)))
    h = _tc_proj(x_p, W_in_p, b_in)

    for l in range(L):
        q, k, v, s_ = _tc_qkvs(h, Wq[l], bq[l], Wk[l], bk[l], Wv[l], bv[l], Ws[l], bs[l])
        nums = []
        dens = []
        for b in range(B):
            qt = q[b].reshape(HEADS * NP, DH)
            kt = k[b].reshape(HEADS * NP, DH)
            vt = v[b].reshape(HEADS * NP, DH)
            numb, denb = _sc_attn(qt, kt, vt, srcp, dstp)
            num_t = numb.reshape(HEADS, NP, DH)[:, :N].transpose(1, 0, 2).reshape(N, H)
            den_bn = denb.reshape(HEADS, NP)[:, :N]
            den_r = jnp.repeat(den_bn.T, DH, axis=1)
            nums.append(num_t)
            dens.append(den_r)
        num_t = jnp.stack(nums)
        den_r = jnp.stack(dens)
        h = _tc_combine(num_t, den_r, s_, h, gamma, beta)

    P = W_out.shape[1]
    Wp = jnp.zeros((H, 128), jnp.float32).at[:, :P].set(W_out)
    bp = jnp.zeros((128,), jnp.float32).at[:P].set(b_out)
    out = _tc_proj(h, Wp, bp)
    return jnp.transpose(out[:, :, :P], (0, 2, 1))


# speedup vs baseline: 1.1228x; 1.0001x over previous
import functools
import jax
import jax.numpy as jnp
from jax import lax
from jax.experimental import pallas as pl
from jax.experimental.pallas import tpu as pltpu
from jax.experimental.pallas import tpu_sc as plsc

N = 50000
NP = 50048
E0 = 800000
C = 128
NTILES = 16
NCH = 392
EP = NTILES * C * NCH
EPX = EP + NTILES * C
HEADS = 4
DH = 16
H = 64
RPT = NP // NTILES
ZR = RPT // 8


HCH = NCH // 2


def _sc_body(qt, kt, vt, src_h, dst_h, num_h, den_h,
             sbig, dbig, gqA, gkA, gqB, gkB,
             qbA, kbA, vbA, qbB, kbB, vbB, mbA, exbA, mbB, exbB, zb, zd,
             num_sp, den_sp, semA0, semA1, semA2, semB0, semB1, semB2,
             semA3, semB3):
    c = lax.axis_index("c")
    s = lax.axis_index("s")
    zero16 = jnp.zeros((16,), jnp.float32)
    iota16 = lax.iota(jnp.int32, 16)
    row0 = s * RPT

    def zrow(i, _):
        zb[i] = zero16
        return 0
    lax.fori_loop(0, ZR, zrow, 0)

    def zrowd(i, _):
        zd[pl.ds(i * 16, 16)] = zero16
        return 0
    lax.fori_loop(0, RPT // 16, zrowd, 0)

    for hl in range(2):
        hoff = (2 * c + hl) * NP
        for j in range(8):
            pltpu.sync_copy(zb, num_sp.at[pl.ds(row0 + j * ZR, ZR)])
        pltpu.sync_copy(zd, den_sp.at[pl.ds(row0, RPT)])
        plsc.subcore_barrier()

        def issue(j, gq, gk, qb, kb, vb, s0, s1, s2):
            def gidx(g, _):
                sl = pl.ds(g * 16, 16)
                gq[sl] = dbig[j, sl] + hoff
                gk[sl] = sbig[j, sl] + hoff
                return 0
            lax.fori_loop(0, C // 16, gidx, 0)
            return (pltpu.async_copy(qt.at[gq], qb, s0),
                    pltpu.async_copy(kt.at[gk], kb, s1),
                    pltpu.async_copy(vt.at[gk], vb, s2))

        def compute(j, i, qb, kb, vb, mb, exb, sems, cps):
            for cp in cps:
                cp.wait()

            @pl.when(i != 0)
            def _():
                pltpu.make_async_copy(mb, num_sp.at[dbig.at[j]], sems).wait()
                pltpu.make_async_copy(exb, den_sp.at[dbig.at[j]], sems).wait()

            def grp(g, _):
                av = jnp.zeros((16,), jnp.float32)
                for e in range(16):
                    al = jnp.sum(qb[g * 16 + e] * kb[g * 16 + e])
                    av = jnp.where(iota16 == e, al, av)
                ex = jnp.exp(av * 0.25)
                exb[pl.ds(g * 16, 16)] = ex
                for e in range(16):
                    r = g * 16 + e
                    mb[r] = jnp.full((16,), ex[e], jnp.float32) * vb[r]
                return 0
            lax.fori_loop(0, C // 16, grp, 0)
            pltpu.async_copy(mb, num_sp.at[dbig.at[j]], sems, add=True)
            pltpu.async_copy(exb, den_sp.at[dbig.at[j]], sems, add=True)

        for half in range(2):
            hrow = s * NCH + half * HCH
            pltpu.sync_copy(src_h.at[pl.ds(hrow, HCH + 1)], sbig)
            pltpu.sync_copy(dst_h.at[pl.ds(hrow, HCH + 1)], dbig)

            issue(0, gqA, gkA, qbA, kbA, vbA, semA0, semA1, semA2)

            def pair(i, _):
                cpsB = issue(2 * i + 1, gqB, gkB, qbB, kbB, vbB,
                             semB0, semB1, semB2)
                compute(2 * i, i, qbA, kbA, vbA, mbA, exbA, semA3,
                        (pltpu.make_async_copy(qt.at[gqA], qbA, semA0),
                         pltpu.make_async_copy(kt.at[gkA], kbA, semA1),
                         pltpu.make_async_copy(vt.at[gkA], vbA, semA2)))
                issue(2 * i + 2, gqA, gkA, qbA, kbA, vbA, semA0, semA1, semA2)
                compute(2 * i + 1, i, qbB, kbB, vbB, mbB, exbB, semB3, cpsB)
                return 0
            lax.fori_loop(0, HCH // 2, pair, 0)

            pltpu.make_async_copy(qt.at[gqA], qbA, semA0).wait()
            pltpu.make_async_copy(kt.at[gkA], kbA, semA1).wait()
            pltpu.make_async_copy(vt.at[gkA], vbA, semA2).wait()
            pltpu.make_async_copy(mbA, num_sp.at[dbig.at[0]], semA3).wait()
            pltpu.make_async_copy(exbA, den_sp.at[dbig.at[0]], semA3).wait()
            pltpu.make_async_copy(mbB, num_sp.at[dbig.at[0]], semB3).wait()
            pltpu.make_async_copy(exbB, den_sp.at[dbig.at[0]], semB3).wait()
        plsc.subcore_barrier()

        out0 = hoff + row0
        pltpu.sync_copy(num_sp.at[pl.ds(row0, RPT)], num_h.at[pl.ds(out0, RPT)])
        pltpu.sync_copy(den_sp.at[pl.ds(row0, RPT)], den_h.at[pl.ds(out0, RPT)])


_sc_attn = functools.partial(
    pl.kernel,
    mesh=plsc.VectorSubcoreMesh(core_axis_name="c", subcore_axis_name="s"),
    compiler_params=pltpu.CompilerParams(
        needs_layout_passes=False, use_tc_tiling_on_sc=False),
    out_type=(
        jax.ShapeDtypeStruct((HEADS * NP, DH), jnp.float32),
        jax.ShapeDtypeStruct((HEADS * NP,), jnp.float32),
    ),
    scratch_types=[
        pltpu.VMEM((HCH + 1, C), jnp.int32),
        pltpu.VMEM((HCH + 1, C), jnp.int32),
        pltpu.VMEM((C,), jnp.int32),
        pltpu.VMEM((C,), jnp.int32),
        pltpu.VMEM((C,), jnp.int32),
        pltpu.VMEM((C,), jnp.int32),
        pltpu.VMEM((C, DH), jnp.float32),
        pltpu.VMEM((C, DH), jnp.float32),
        pltpu.VMEM((C, DH), jnp.float32),
        pltpu.VMEM((C, DH), jnp.float32),
        pltpu.VMEM((C, DH), jnp.float32),
        pltpu.VMEM((C, DH), jnp.float32),
        pltpu.VMEM((C, DH), jnp.float32),
        pltpu.VMEM((C,), jnp.float32),
        pltpu.VMEM((C, DH), jnp.float32),
        pltpu.VMEM((C,), jnp.float32),
        pltpu.VMEM((ZR, DH), jnp.float32),
        pltpu.VMEM((RPT,), jnp.float32),
        pltpu.VMEM_SHARED((NP, DH), jnp.float32),
        pltpu.VMEM_SHARED((NP,), jnp.float32),
        pltpu.SemaphoreType.DMA,
        pltpu.SemaphoreType.DMA,
        pltpu.SemaphoreType.DMA,
        pltpu.SemaphoreType.DMA,
        pltpu.SemaphoreType.DMA,
        pltpu.SemaphoreType.DMA,
        pltpu.SemaphoreType.DMA,
        pltpu.SemaphoreType.DMA,
    ],
)(_sc_body)


BLK = 1000


def _proj_body(x_ref, w_ref, b_ref, o_ref):
    o_ref[0] = jnp.dot(x_ref[0], w_ref[...], preferred_element_type=jnp.float32) + b_ref[...]


def _tc_proj(xp, w, b):
    B, n, kdim = xp.shape
    hdim = w.shape[1]
    return pl.pallas_call(
        _proj_body,
        grid=(B, n // BLK),
        in_specs=[
            pl.BlockSpec((1, BLK, kdim), lambda b_, i: (b_, i, 0)),
            pl.BlockSpec((kdim, hdim), lambda b_, i: (0, 0)),
            pl.BlockSpec((hdim,), lambda b_, i: (0,)),
        ],
        out_specs=pl.BlockSpec((1, BLK, hdim), lambda b_, i: (b_, i, 0)),
        out_shape=jax.ShapeDtypeStruct((B, n, hdim), jnp.float32),
    )(xp, w, b)


def _qkvs_body(h_ref, wq_ref, bq_ref, wk_ref, bk_ref, wv_ref, bv_ref, ws_ref, bs_ref,
               q_ref, k_ref, v_ref, s_ref):
    hb = h_ref[0]
    q = jnp.dot(hb, wq_ref[...], preferred_element_type=jnp.float32) + bq_ref[...]
    k = jnp.dot(hb, wk_ref[...], preferred_element_type=jnp.float32) + bk_ref[...]
    v = jnp.dot(hb, wv_ref[...], preferred_element_type=jnp.float32) + bv_ref[...]
    for hh in range(HEADS):
        q_ref[0, hh] = q[:, hh * DH:(hh + 1) * DH]
        k_ref[0, hh] = k[:, hh * DH:(hh + 1) * DH]
        v_ref[0, hh] = v[:, hh * DH:(hh + 1) * DH]
    s_ref[0] = jnp.dot(hb, ws_ref[...], preferred_element_type=jnp.float32) + bs_ref[...]


def _tc_qkvs(h, wq, bq, wk, bk, wv, bv, ws, bs):
    B, n, _ = h.shape
    wspec = pl.BlockSpec((H, H), lambda b_, i: (0, 0))
    bspec = pl.BlockSpec((H,), lambda b_, i: (0,))
    hspec = pl.BlockSpec((1, BLK, H), lambda b_, i: (b_, i, 0))
    tspec = pl.BlockSpec((1, HEADS, BLK, DH), lambda b_, i: (b_, 0, i, 0))
    t_sds = jax.ShapeDtypeStruct((B, HEADS, NP, DH), jnp.float32)
    out_sds = jax.ShapeDtypeStruct((B, n, H), jnp.float32)
    return pl.pallas_call(
        _qkvs_body,
        grid=(B, n // BLK),
        in_specs=[hspec, wspec, bspec, wspec, bspec, wspec, bspec, wspec, bspec],
        out_specs=(tspec, tspec, tspec, hspec),
        out_shape=(t_sds, t_sds, t_sds, out_sds),
    )(h, wq, bq, wk, bk, wv, bv, ws, bs)


def _combine_body(num_ref, den_ref, s_ref, h_ref, g_ref, b_ref, o_ref):
    a = num_ref[0] / jnp.maximum(den_ref[0], 1e-16) + s_ref[0] + h_ref[0]
    mu = jnp.mean(a, axis=-1, keepdims=True)
    var = jnp.mean((a - mu) * (a - mu), axis=-1, keepdims=True)
    o_ref[0] = (a - mu) / jnp.sqrt(var + 1e-5) * g_ref[...] + b_ref[...]


def _tc_combine(num_t, den_r, s_, h, gamma, beta):
    B, n, _ = h.shape
    hspec = pl.BlockSpec((1, BLK, H), lambda b_, i: (b_, i, 0))
    vspec = pl.BlockSpec((H,), lambda b_, i: (0,))
    return pl.pallas_call(
        _combine_body,
        grid=(B, n // BLK),
        in_specs=[hspec, hspec, hspec, hspec, vspec, vspec],
        out_specs=hspec,
        out_shape=jax.ShapeDtypeStruct((B, n, H), jnp.float32),
    )(num_t, den_r, s_, h, gamma, beta)


def kernel(x, edge_index, W_in, b_in, Wq, bq, Wk, bk, Wv, bv, Ws, bs, gamma, beta, W_out, b_out):
    B = x.shape[0]
    S = x.shape[1]
    L = Wq.shape[0]
    src = edge_index[0]
    dst = edge_index[1]
    srcp = jnp.full((EPX,), N, jnp.int32).at[:E0].set(src).reshape(EPX // C, C)
    dstp = jnp.full((EPX,), N, jnp.int32).at[:E0].set(dst).reshape(EPX // C, C)

    x_t = jnp.transpose(x, (0, 2, 1))
    x_p = jnp.pad(x_t, ((0, 0), (0, 0), (0, 16 - S)))
    W_in_p = jnp.pad(W_in, ((0, 16 - S), (0, 0)))
    h = _tc_proj(x_p, W_in_p, b_in)

    for l in range(L):
        q, k, v, s_ = _tc_qkvs(h, Wq[l], bq[l], Wk[l], bk[l], Wv[l], bv[l], Ws[l], bs[l])
        nums = []
        dens = []
        for b in range(B):
            qt = q[b].reshape(HEADS * NP, DH)
            kt = k[b].reshape(HEADS * NP, DH)
            vt = v[b].reshape(HEADS * NP, DH)
            numb, denb = _sc_attn(qt, kt, vt, srcp, dstp)
            num_t = numb.reshape(HEADS, NP, DH)[:, :N].transpose(1, 0, 2).reshape(N, H)
            den_bn = denb.reshape(HEADS, NP)[:, :N]
            den_r = jnp.repeat(den_bn.T, DH, axis=1)
            nums.append(num_t)
            dens.append(den_r)
        num_t = jnp.stack(nums)
        den_r = jnp.stack(dens)
        h = _tc_combine(num_t, den_r, s_, h, gamma, beta)

    P = W_out.shape[1]
    Wp = jnp.zeros((H, 128), jnp.float32).at[:, :P].set(W_out)
    bp = jnp.zeros((128,), jnp.float32).at[:P].set(b_out)
    out = _tc_proj(h, Wp, bp)
    return jnp.transpose(out[:, :, :P], (0, 2, 1))
